# Initial kernel scaffold; baseline (speedup 1.0000x reference)
import jax
import jax.numpy as jnp
from jax.experimental import pallas as pl


def kernel(user_inputs, u_item_inputs, u2e, i2e):
    raise NotImplementedError("write your pallas kernel here")


# speedup vs baseline: 1.1579x; 1.1579x over previous
import functools

import jax
import jax.numpy as jnp
from jax import lax
from jax.experimental import pallas as pl
from jax.experimental.pallas import tpu as pltpu
from jax.experimental.pallas import tpu_sc as plsc

NC = 2
NS = 16
NW = NC * NS

BATCH = 16384
D = 128
B_PER_W = BATCH // NW
CHUNK = 128
NCHUNK = B_PER_W // CHUNK
GROUPS = CHUNK // 16


def _sc_body(uidx_hbm, iidx_hbm, u2e_hbm, i2e_hbm, out_hbm,
             uidx_v, iidx_v, u_rows, i_rows, out_v, sem_u, sem_i):
    wid = lax.axis_index("s") * NC + lax.axis_index("c")
    base = wid * B_PER_W

    pltpu.sync_copy(uidx_hbm.at[wid], uidx_v)
    pltpu.sync_copy(iidx_hbm.at[wid], iidx_v)

    lane = lax.iota(jnp.int32, 16)

    for c in range(NCHUNK):
        cu = pltpu.async_copy(u2e_hbm.at[uidx_v.at[c]], u_rows, sem_u)
        ci = pltpu.async_copy(i2e_hbm.at[iidx_v.at[c]], i_rows, sem_i)
        cu.wait()
        ci.wait()

        for g in range(GROUPS):
            def rbody(i, res, _g=g):
                r = _g * 16 + i
                acc = u_rows[r, pl.ds(0, 16)] * i_rows[r, pl.ds(0, 16)]
                for j in range(1, D // 16):
                    acc += (u_rows[r, pl.ds(j * 16, 16)]
                            * i_rows[r, pl.ds(j * 16, 16)])
                for s in (8, 4, 2, 1):
                    perm = jnp.bitwise_xor(lane, s)
                    acc = acc + acc.at[perm].get(mode="promise_in_bounds")
                return jnp.where(lane == i, acc, res)

            res = lax.fori_loop(0, 16, rbody, jnp.zeros((16,), jnp.float32))
            out_v[pl.ds(c * CHUNK + g * 16, 16)] = 1.0 / (1.0 + jnp.exp(-res))

    pltpu.sync_copy(out_v, out_hbm.at[pl.ds(base, B_PER_W)])


@jax.jit
def _run(uidx, iidx, u2e, i2e):
    mesh = plsc.VectorSubcoreMesh(core_axis_name="c", subcore_axis_name="s")
    f = pl.kernel(
        _sc_body,
        mesh=mesh,
        out_type=jax.ShapeDtypeStruct((BATCH,), jnp.float32),
        scratch_types=[
            pltpu.VMEM((NCHUNK, CHUNK), jnp.int32),
            pltpu.VMEM((NCHUNK, CHUNK), jnp.int32),
            pltpu.VMEM((CHUNK, D), jnp.float32),
            pltpu.VMEM((CHUNK, D), jnp.float32),
            pltpu.VMEM((B_PER_W,), jnp.float32),
            pltpu.SemaphoreType.DMA,
            pltpu.SemaphoreType.DMA,
        ],
    )
    return f(uidx, iidx, u2e, i2e)


def kernel(user_inputs, u_item_inputs, u2e, i2e):
    uidx = user_inputs.reshape(NW, NCHUNK, CHUNK)
    iidx = u_item_inputs.reshape(NW, NCHUNK, CHUNK)
    return _run(uidx, iidx, u2e, i2e)
